# Initial kernel scaffold; baseline (speedup 1.0000x reference)
import jax
import jax.numpy as jnp
from jax.experimental import pallas as pl


def kernel(input, W):
    raise NotImplementedError("write your pallas kernel here")


# speedup vs baseline: 1.0376x; 1.0376x over previous
import jax
import jax.numpy as jnp
from jax.experimental import pallas as pl
from jax.experimental.pallas import tpu as pltpu


def _router_body(x_ref, w_ref, idx_ref, loc_ref, gate_ref, laux_ref,
                 cnt_ref, me_ref):
    i = pl.program_id(0)
    nblocks = pl.num_programs(0)
    B = x_ref.shape[0]
    E = w_ref.shape[0]

    @pl.when(i == 0)
    def _init():
        cnt_ref[...] = jnp.zeros_like(cnt_ref)
        me_ref[...] = jnp.zeros_like(me_ref)

    x = x_ref[...]
    w = w_ref[...]
    logits = jax.lax.dot_general(
        x, w, (((1,), (1,)), ((), ())),
        preferred_element_type=jnp.float32)

    rowmax = jnp.max(logits, axis=1, keepdims=True)
    e = jnp.exp(logits - rowmax)
    s = jnp.sum(e, axis=1, keepdims=True)
    gate_ref[...] = 1.0 / s

    lane = jax.lax.broadcasted_iota(jnp.int32, (B, E), 1)
    idx = jnp.min(jnp.where(logits == rowmax, lane, E), axis=1,
                  keepdims=True)
    idx_ref[...] = idx

    mask = (lane == idx).astype(jnp.float32)

    r = jax.lax.broadcasted_iota(jnp.int32, (B, B), 0)
    c = jax.lax.broadcasted_iota(jnp.int32, (B, B), 1)
    tri = (c < r).astype(jnp.float32)
    pref = jax.lax.dot_general(
        tri, mask, (((1,), (0,)), ((), ())),
        preferred_element_type=jnp.float32)
    carry = cnt_ref[...]
    loc = (jnp.sum(pref * mask, axis=1, keepdims=True)
           + jnp.sum(carry * mask, axis=1, keepdims=True))
    loc_ref[...] = loc.astype(jnp.int32)

    cnt_ref[...] = carry + jnp.sum(mask, axis=0, keepdims=True)
    me_ref[...] = me_ref[...] + jnp.sum(e / s, axis=0, keepdims=True)

    @pl.when(i == nblocks - 1)
    def _fin():
        n_tok = B * nblocks
        ce = cnt_ref[...]
        me = me_ref[...]
        laux_ref[...] = (jnp.sum(me * ce, keepdims=True)
                         * (E / (n_tok * n_tok))).reshape(1, 1)


def kernel(input, W):
    N, D = input.shape
    E = W.shape[0]
    B = 256
    nblocks = N // B
    capacity = int(1.0 * ((N + E - 1) // E))

    idx, loc, gate, laux = pl.pallas_call(
        _router_body,
        grid=(nblocks,),
        in_specs=[
            pl.BlockSpec((B, D), lambda i: (i, 0)),
            pl.BlockSpec((E, D), lambda i: (0, 0)),
        ],
        out_specs=[
            pl.BlockSpec((B, 1), lambda i: (i, 0)),
            pl.BlockSpec((B, 1), lambda i: (i, 0)),
            pl.BlockSpec((B, 1), lambda i: (i, 0)),
            pl.BlockSpec((1, 1), lambda i: (0, 0)),
        ],
        out_shape=[
            jax.ShapeDtypeStruct((N, 1), jnp.int32),
            jax.ShapeDtypeStruct((N, 1), jnp.int32),
            jax.ShapeDtypeStruct((N, 1), jnp.float32),
            jax.ShapeDtypeStruct((1, 1), jnp.float32),
        ],
        scratch_shapes=[
            pltpu.VMEM((1, E), jnp.float32),
            pltpu.VMEM((1, E), jnp.float32),
        ],
    )(input, W)

    return (laux[0, 0],
            jnp.asarray(capacity, dtype=jnp.int32),
            jnp.asarray(E, dtype=jnp.int32),
            idx[:, 0],
            loc[:, 0],
            gate[:, 0])


# speedup vs baseline: 1.1974x; 1.1541x over previous
import jax
import jax.numpy as jnp
from jax.experimental import pallas as pl
from jax.experimental.pallas import tpu as pltpu


def _router_body(x_ref, w_ref, idx_ref, loc_ref, gate_ref, laux_ref,
                 cnt_ref, me_ref):
    i = pl.program_id(0)
    nblocks = pl.num_programs(0)
    B = x_ref.shape[0]
    E = w_ref.shape[0]

    @pl.when(i == 0)
    def _init():
        cnt_ref[...] = jnp.zeros_like(cnt_ref)
        me_ref[...] = jnp.zeros_like(me_ref)

    x = x_ref[...]
    w = w_ref[...]
    logits = jax.lax.dot_general(
        x, w, (((1,), (1,)), ((), ())),
        preferred_element_type=jnp.float32)

    rowmax = jnp.max(logits, axis=1, keepdims=True)
    e = jnp.exp(logits - rowmax)
    s = jnp.sum(e, axis=1, keepdims=True)
    gate_ref[...] = 1.0 / s

    lane = jax.lax.broadcasted_iota(jnp.int32, (B, E), 1)
    idx = jnp.min(jnp.where(logits == rowmax, lane, E), axis=1,
                  keepdims=True)
    idx_ref[...] = idx

    mask = (lane == idx).astype(jnp.float32)

    r = jax.lax.broadcasted_iota(jnp.int32, (B, B), 0)
    c = jax.lax.broadcasted_iota(jnp.int32, (B, B), 1)
    tri = (c < r).astype(jnp.float32)
    pref = jax.lax.dot_general(
        tri, mask, (((1,), (0,)), ((), ())),
        preferred_element_type=jnp.float32)
    carry = cnt_ref[...]
    loc = (jnp.sum(pref * mask, axis=1, keepdims=True)
           + jnp.sum(carry * mask, axis=1, keepdims=True))
    loc_ref[...] = loc.astype(jnp.int32)

    cnt_ref[...] = carry + jnp.sum(mask, axis=0, keepdims=True)
    me_ref[...] = me_ref[...] + jnp.sum(e / s, axis=0, keepdims=True)

    @pl.when(i == nblocks - 1)
    def _fin():
        n_tok = B * nblocks
        ce = cnt_ref[...]
        me = me_ref[...]
        laux_ref[...] = (jnp.sum(me * ce, keepdims=True)
                         * (E / (n_tok * n_tok))).reshape(1, 1)


def kernel(input, W):
    N, D = input.shape
    E = W.shape[0]
    B = 512
    nblocks = N // B
    capacity = int(1.0 * ((N + E - 1) // E))

    idx, loc, gate, laux = pl.pallas_call(
        _router_body,
        grid=(nblocks,),
        in_specs=[
            pl.BlockSpec((B, D), lambda i: (i, 0)),
            pl.BlockSpec((E, D), lambda i: (0, 0)),
        ],
        out_specs=[
            pl.BlockSpec((B, 1), lambda i: (i, 0)),
            pl.BlockSpec((B, 1), lambda i: (i, 0)),
            pl.BlockSpec((B, 1), lambda i: (i, 0)),
            pl.BlockSpec((1, 1), lambda i: (0, 0)),
        ],
        out_shape=[
            jax.ShapeDtypeStruct((N, 1), jnp.int32),
            jax.ShapeDtypeStruct((N, 1), jnp.int32),
            jax.ShapeDtypeStruct((N, 1), jnp.float32),
            jax.ShapeDtypeStruct((1, 1), jnp.float32),
        ],
        scratch_shapes=[
            pltpu.VMEM((1, E), jnp.float32),
            pltpu.VMEM((1, E), jnp.float32),
        ],
    )(input, W)

    return (laux[0, 0],
            jnp.asarray(capacity, dtype=jnp.int32),
            jnp.asarray(E, dtype=jnp.int32),
            idx[:, 0],
            loc[:, 0],
            gate[:, 0])


# speedup vs baseline: 1.2318x; 1.0287x over previous
import jax
import jax.numpy as jnp
from jax.experimental import pallas as pl
from jax.experimental.pallas import tpu as pltpu


def _router_body(x_ref, w_ref, idx_ref, loc_ref, gate_ref, laux_ref,
                 cnt_ref, me_ref):
    i = pl.program_id(0)
    nblocks = pl.num_programs(0)
    B = x_ref.shape[0]
    E = w_ref.shape[0]

    @pl.when(i == 0)
    def _init():
        cnt_ref[...] = jnp.zeros_like(cnt_ref)
        me_ref[...] = jnp.zeros_like(me_ref)

    x = x_ref[...]
    w = w_ref[...]
    logits = jax.lax.dot_general(
        x, w, (((1,), (1,)), ((), ())),
        preferred_element_type=jnp.float32)

    rowmax = jnp.max(logits, axis=1, keepdims=True)
    e = jnp.exp(logits - rowmax)
    s = jnp.sum(e, axis=1, keepdims=True)
    gate_ref[...] = 1.0 / s

    lane = jax.lax.broadcasted_iota(jnp.int32, (B, E), 1)
    idx = jnp.min(jnp.where(logits == rowmax, lane, E), axis=1,
                  keepdims=True)
    idx_ref[...] = idx

    mask = (lane == idx).astype(jnp.float32)

    r = jax.lax.broadcasted_iota(jnp.int32, (B, B), 0)
    c = jax.lax.broadcasted_iota(jnp.int32, (B, B), 1)
    tri = (c < r).astype(jnp.float32)
    pref = jax.lax.dot_general(
        tri, mask, (((1,), (0,)), ((), ())),
        preferred_element_type=jnp.float32)
    carry = cnt_ref[...]
    loc = (jnp.sum(pref * mask, axis=1, keepdims=True)
           + jnp.sum(carry * mask, axis=1, keepdims=True))
    loc_ref[...] = loc.astype(jnp.int32)

    cnt_ref[...] = carry + jnp.sum(mask, axis=0, keepdims=True)
    me_ref[...] = me_ref[...] + jnp.sum(e / s, axis=0, keepdims=True)

    @pl.when(i == nblocks - 1)
    def _fin():
        n_tok = B * nblocks
        ce = cnt_ref[...]
        me = me_ref[...]
        laux_ref[...] = (jnp.sum(me * ce, keepdims=True)
                         * (E / (n_tok * n_tok))).reshape(1, 1)


def kernel(input, W):
    N, D = input.shape
    E = W.shape[0]
    B = 1024
    nblocks = N // B
    capacity = int(1.0 * ((N + E - 1) // E))

    idx, loc, gate, laux = pl.pallas_call(
        _router_body,
        grid=(nblocks,),
        in_specs=[
            pl.BlockSpec((B, D), lambda i: (i, 0)),
            pl.BlockSpec((E, D), lambda i: (0, 0)),
        ],
        out_specs=[
            pl.BlockSpec((B, 1), lambda i: (i, 0)),
            pl.BlockSpec((B, 1), lambda i: (i, 0)),
            pl.BlockSpec((B, 1), lambda i: (i, 0)),
            pl.BlockSpec((1, 1), lambda i: (0, 0)),
        ],
        out_shape=[
            jax.ShapeDtypeStruct((N, 1), jnp.int32),
            jax.ShapeDtypeStruct((N, 1), jnp.int32),
            jax.ShapeDtypeStruct((N, 1), jnp.float32),
            jax.ShapeDtypeStruct((1, 1), jnp.float32),
        ],
        scratch_shapes=[
            pltpu.VMEM((1, E), jnp.float32),
            pltpu.VMEM((1, E), jnp.float32),
        ],
    )(input, W)

    return (laux[0, 0],
            jnp.asarray(capacity, dtype=jnp.int32),
            jnp.asarray(E, dtype=jnp.int32),
            idx[:, 0],
            loc[:, 0],
            gate[:, 0])
